# baseline (speedup 1.0000x reference)
import functools

import jax
import jax.numpy as jnp
from jax import lax
from jax.experimental import pallas as pl
from jax.experimental.pallas import tpu as pltpu
from jax.experimental.pallas import tpu_sc as plsc

_B = 4096
_K = 8192
_D = 64
_BM = 256


def _argmin_body(x_ref, lm_ref, a2_ref, b2_ref, idx_ref):
    x = x_ref[...]
    lm = lm_ref[...]
    s = lax.dot_general(x, lm, (((1,), (1,)), ((), ())),
                        preferred_element_type=jnp.float32)
    d2 = (a2_ref[...] + b2_ref[...]) - 2.0 * s
    d = jnp.sqrt(jnp.maximum(d2, 0.0))
    row_min = jnp.min(d, axis=1, keepdims=True)
    iota = lax.broadcasted_iota(jnp.int32, d.shape, 1)
    idx = jnp.min(jnp.where(d == row_min, iota, _K), axis=1)
    idx_ref[...] = idx.astype(jnp.int32)


def _argmin_tc(x, landmarks, a2, b2):
    grid = (_B // _BM,)
    return pl.pallas_call(
        _argmin_body,
        grid=grid,
        in_specs=[
            pl.BlockSpec((_BM, _D), lambda i: (i, 0)),
            pl.BlockSpec((_K, _D), lambda i: (0, 0)),
            pl.BlockSpec((_BM, 1), lambda i: (i, 0)),
            pl.BlockSpec((1, _K), lambda i: (0, 0)),
        ],
        out_specs=pl.BlockSpec((_BM,), lambda i: (i,)),
        out_shape=jax.ShapeDtypeStruct((_B,), jnp.int32),
        compiler_params=pltpu.CompilerParams(
            dimension_semantics=("parallel",)),
    )(x, landmarks, a2, b2)


_DP = 128


def _gather_sc(table_pad, idx):
    info = plsc.get_sparse_core_info()
    nw = info.num_cores * info.num_subcores
    b_per_w = _B // nw
    mesh = plsc.VectorSubcoreMesh(core_axis_name="c", subcore_axis_name="s")

    @functools.partial(
        pl.kernel, mesh=mesh,
        out_type=jax.ShapeDtypeStruct((_B, _DP), jnp.float32),
        scratch_types=[
            pltpu.VMEM((b_per_w,), jnp.int32),
            pltpu.VMEM((b_per_w, _DP), jnp.float32),
            pltpu.SemaphoreType.DMA,
        ],
    )
    def k(table_hbm, idx_hbm, out_hbm, idx_v, rows_v, sem):
        wid = lax.axis_index("s") * info.num_cores + lax.axis_index("c")
        base = wid * b_per_w
        pltpu.sync_copy(idx_hbm.at[pl.ds(base, b_per_w)], idx_v)
        pltpu.async_copy(table_hbm.at[idx_v], rows_v, sem).wait()
        pltpu.sync_copy(rows_v, out_hbm.at[pl.ds(base, b_per_w)])

    return k(table_pad, idx)


def kernel(x, landmarks):
    a2 = jnp.sum(x * x, axis=-1, keepdims=True)
    b2 = jnp.sum(landmarks * landmarks, axis=-1)[None, :]
    idx = _argmin_tc(x, landmarks, a2, b2)
    table_pad = jnp.pad(landmarks, ((0, 0), (0, _DP - _D)))
    return _gather_sc(table_pad, idx)[:, :_D]


# speedup vs baseline: 1.4353x; 1.4353x over previous
import functools

import jax
import jax.numpy as jnp
from jax import lax
from jax.experimental import pallas as pl
from jax.experimental.pallas import tpu as pltpu
from jax.experimental.pallas import tpu_sc as plsc

_B = 4096
_K = 8192
_D = 64
_BM = 256


def _argmin_body(xm2_ref, lm_ref, a2_ref, b2_ref, idx_ref):
    xm2 = xm2_ref[...]
    lm = lm_ref[...]
    s = lax.dot_general(xm2, lm, (((1,), (1,)), ((), ())),
                        preferred_element_type=jnp.float32)
    d2 = (a2_ref[...] + b2_ref[...]) + s
    rm2 = jnp.min(d2, axis=1, keepdims=True)
    r = jnp.sqrt(jnp.maximum(rm2, 0.0))
    ib = lax.bitcast_convert_type(r * r, jnp.int32)
    karr = lax.broadcasted_iota(jnp.int32, (1, 14), 1) - 6
    cks = lax.bitcast_convert_type(ib + karr, jnp.float32)
    oks = jnp.sqrt(jnp.maximum(cks, 0.0)) <= r
    t = jnp.max(jnp.where(oks, cks, rm2), axis=1, keepdims=True)
    t = jnp.where(rm2 <= 0.0, 0.0, t)
    fm = jnp.full((d2.shape[0], 128), 64, jnp.int32)
    for f in range(63, -1, -1):
        fm = jnp.where(d2[:, f * 128:(f + 1) * 128] <= t, f, fm)
    z = fm * 128 + lax.broadcasted_iota(jnp.int32, fm.shape, 1)
    idx_ref[...] = jnp.min(z, axis=1)


def _argmin_tc(x, landmarks, a2, b2):
    grid = (_B // _BM,)
    return pl.pallas_call(
        _argmin_body,
        grid=grid,
        in_specs=[
            pl.BlockSpec((_BM, _D), lambda i: (i, 0)),
            pl.BlockSpec((_K, _D), lambda i: (0, 0)),
            pl.BlockSpec((_BM, 1), lambda i: (i, 0)),
            pl.BlockSpec((1, _K), lambda i: (0, 0)),
        ],
        out_specs=pl.BlockSpec((_BM,), lambda i: (i,)),
        out_shape=jax.ShapeDtypeStruct((_B,), jnp.int32),
        compiler_params=pltpu.CompilerParams(
            dimension_semantics=("parallel",)),
    )(x, landmarks, a2, b2)


_DP = 128


def _gather_sc(table_pad, idx):
    info = plsc.get_sparse_core_info()
    nw = info.num_cores * info.num_subcores
    b_per_w = _B // nw
    mesh = plsc.VectorSubcoreMesh(core_axis_name="c", subcore_axis_name="s")

    @functools.partial(
        pl.kernel, mesh=mesh,
        out_type=jax.ShapeDtypeStruct((_B, _DP), jnp.float32),
        scratch_types=[
            pltpu.VMEM((b_per_w,), jnp.int32),
            pltpu.VMEM((b_per_w, _DP), jnp.float32),
            pltpu.SemaphoreType.DMA,
        ],
    )
    def k(table_hbm, idx_hbm, out_hbm, idx_v, rows_v, sem):
        wid = lax.axis_index("s") * info.num_cores + lax.axis_index("c")
        base = wid * b_per_w
        pltpu.sync_copy(idx_hbm.at[pl.ds(base, b_per_w)], idx_v)
        pltpu.async_copy(table_hbm.at[idx_v], rows_v, sem).wait()
        pltpu.sync_copy(rows_v, out_hbm.at[pl.ds(base, b_per_w)])

    return k(table_pad, idx)


def kernel(x, landmarks):
    a2 = jnp.sum(x * x, axis=-1, keepdims=True)
    b2 = jnp.sum(landmarks * landmarks, axis=-1)[None, :]
    idx = _argmin_tc(x * -2.0, landmarks, a2, b2)
    table_pad = jnp.pad(landmarks, ((0, 0), (0, _DP - _D)))
    return _gather_sc(table_pad, idx)[:, :_D]


# speedup vs baseline: 1.5427x; 1.0749x over previous
import functools

import jax
import jax.numpy as jnp
from jax import lax
from jax.experimental import pallas as pl
from jax.experimental.pallas import tpu as pltpu
from jax.experimental.pallas import tpu_sc as plsc

_B = 4096
_K = 8192
_D = 64
_BM = 512
_DP = 128


def _argmin_body(xm2_ref, lm_ref, a2_ref, b2_ref, idx_ref, tp_ref):
    xm2 = xm2_ref[...]
    lm = lm_ref[...]
    s = lax.dot_general(xm2, lm, (((1,), (1,)), ((), ())),
                        preferred_element_type=jnp.float32)
    d2 = (a2_ref[...] + b2_ref[...]) + s
    rm2 = jnp.min(d2, axis=1, keepdims=True)
    r = jnp.sqrt(jnp.maximum(rm2, 0.0))
    ib = lax.bitcast_convert_type(r * r, jnp.int32)
    karr = lax.broadcasted_iota(jnp.int32, (1, 14), 1) - 6
    cks = lax.bitcast_convert_type(ib + karr, jnp.float32)
    oks = jnp.sqrt(jnp.maximum(cks, 0.0)) <= r
    t = jnp.max(jnp.where(oks, cks, rm2), axis=1, keepdims=True)
    t = jnp.where(rm2 <= 0.0, 0.0, t)
    fm = jnp.full((d2.shape[0], 128), 64, jnp.int32)
    for f in range(63, -1, -1):
        fm = jnp.where(d2[:, f * 128:(f + 1) * 128] <= t, f, fm)
    z = fm * 128 + lax.broadcasted_iota(jnp.int32, fm.shape, 1)
    idx_ref[...] = jnp.min(z, axis=1)
    kb = tp_ref.shape[0]
    rows = lm_ref[pl.ds(pl.program_id(0) * kb, kb), :]
    tp_ref[...] = jnp.concatenate(
        [rows, jnp.zeros((kb, _DP - _D), jnp.float32)], axis=1)


def _argmin_tc(x, landmarks, a2, b2):
    grid = (_B // _BM,)
    return pl.pallas_call(
        _argmin_body,
        grid=grid,
        in_specs=[
            pl.BlockSpec((_BM, _D), lambda i: (i, 0)),
            pl.BlockSpec((_K, _D), lambda i: (0, 0)),
            pl.BlockSpec((_BM, 1), lambda i: (i, 0)),
            pl.BlockSpec((1, _K), lambda i: (0, 0)),
        ],
        out_specs=[
            pl.BlockSpec((_BM,), lambda i: (i,)),
            pl.BlockSpec((_K * _BM // _B, _DP), lambda i: (i, 0)),
        ],
        out_shape=[
            jax.ShapeDtypeStruct((_B,), jnp.int32),
            jax.ShapeDtypeStruct((_K, _DP), jnp.float32),
        ],
        compiler_params=pltpu.CompilerParams(
            dimension_semantics=("parallel",)),
    )(x, landmarks, a2, b2)


def _gather_sc(table_pad, idx):
    info = plsc.get_sparse_core_info()
    nw = info.num_cores * info.num_subcores
    b_per_w = _B // nw
    mesh = plsc.VectorSubcoreMesh(core_axis_name="c", subcore_axis_name="s")

    @functools.partial(
        pl.kernel, mesh=mesh,
        out_type=jax.ShapeDtypeStruct((_B, _DP), jnp.float32),
        scratch_types=[
            pltpu.VMEM((b_per_w,), jnp.int32),
            pltpu.VMEM((b_per_w, _DP), jnp.float32),
            pltpu.SemaphoreType.DMA,
        ],
    )
    def k(table_hbm, idx_hbm, out_hbm, idx_v, rows_v, sem):
        wid = lax.axis_index("s") * info.num_cores + lax.axis_index("c")
        base = wid * b_per_w
        pltpu.sync_copy(idx_hbm.at[pl.ds(base, b_per_w)], idx_v)
        pltpu.async_copy(table_hbm.at[idx_v], rows_v, sem).wait()
        pltpu.sync_copy(rows_v, out_hbm.at[pl.ds(base, b_per_w)])

    return k(table_pad, idx)


def kernel(x, landmarks):
    a2 = jnp.sum(x * x, axis=-1, keepdims=True)
    b2 = jnp.sum(landmarks * landmarks, axis=-1)[None, :]
    idx, table_pad = _argmin_tc(x * -2.0, landmarks, a2, b2)
    return _gather_sc(table_pad, idx)[:, :_D]
